# Initial kernel scaffold; baseline (speedup 1.0000x reference)
import jax
import jax.numpy as jnp
from jax.experimental import pallas as pl


def kernel(x_p, x_d, edge_attr_p, edge_attr_d, x_p_batch, x_d_batch, edge_index_p, edge_index_d, Wp, bp, Wd, bd, lW0, lb0, lW1, lb1):
    raise NotImplementedError("write your pallas kernel here")


# speedup vs baseline: 5.9562x; 5.9562x over previous
import functools
import jax
import jax.numpy as jnp
from jax import lax
from jax.experimental import pallas as pl
from jax.experimental.pallas import tpu as pltpu, tpu_sc as plsc

N = 10000
NP = 10240
E = 160000
EP = 163840
EROWS = EP // 128
STEPS = EP // 128 // 16
RPT = NP // 16
D = 256
B = 64

@functools.cache
def _mesh():
    return plsc.VectorSubcoreMesh(core_axis_name="c", subcore_axis_name="s",
                                  num_cores=2, num_subcores=16)


def _deg_body(dstp_hbm, dstd_hbm, z_hbm, ones_hbm, out_hbm,
              acc_sh, dst_t, ones_v, sem_a, sem_b):
    cid = lax.axis_index("c")
    sid = lax.axis_index("s")
    base = sid * RPT
    pltpu.sync_copy(z_hbm, acc_sh.at[pl.ds(base, RPT)])
    pltpu.sync_copy(ones_hbm, ones_v)

    @pl.when(cid == 0)
    def _():
        pltpu.sync_copy(dstp_hbm.at[pl.ds(sid * STEPS, STEPS)], dst_t)

    @pl.when(cid == 1)
    def _():
        pltpu.sync_copy(dstd_hbm.at[pl.ds(sid * STEPS, STEPS)], dst_t)

    plsc.subcore_barrier()

    def pair(i, carry):
        @pl.when(i > 0)
        def _():
            pltpu.make_async_copy(ones_v, acc_sh.at[dst_t.at[0]], sem_a).wait()
            pltpu.make_async_copy(ones_v, acc_sh.at[dst_t.at[0]], sem_b).wait()

        pltpu.async_copy(ones_v, acc_sh.at[dst_t.at[2 * i]], sem_a, add=True)
        pltpu.async_copy(ones_v, acc_sh.at[dst_t.at[2 * i + 1]], sem_b, add=True)
        return carry

    lax.fori_loop(0, STEPS // 2, pair, 0)
    pltpu.make_async_copy(ones_v, acc_sh.at[dst_t.at[0]], sem_a).wait()
    pltpu.make_async_copy(ones_v, acc_sh.at[dst_t.at[0]], sem_b).wait()
    plsc.subcore_barrier()
    pltpu.sync_copy(acc_sh.at[pl.ds(base, RPT)],
                    out_hbm.at[cid, pl.ds(base, RPT)])


@functools.cache
def _deg_call():
    return pl.kernel(
        _deg_body,
        out_type=jax.ShapeDtypeStruct((2, NP, 128), jnp.float32),
        mesh=_mesh(),
        scratch_types=[
            pltpu.VMEM_SHARED((NP, 128), jnp.float32),
            pltpu.VMEM((STEPS, 128), jnp.int32),
            pltpu.VMEM((128, 128), jnp.float32),
            pltpu.SemaphoreType.DMA,
            pltpu.SemaphoreType.DMA,
        ],
    )


_HALF = STEPS // 2


def _scat_body(g_hbm, src_hbm, dst_hbm, z_hbm, out_hbm,
               acc_sh, src_t, dst_t, r0, r1, sem0, sem1):
    cid = lax.axis_index("c")
    sid = lax.axis_index("s")
    base = sid * RPT
    pltpu.sync_copy(z_hbm, acc_sh.at[pl.ds(base, RPT)])
    plsc.subcore_barrier()

    for phase in range(2):
        off = sid * STEPS + phase * _HALF
        pltpu.sync_copy(src_hbm.at[cid, pl.ds(off, _HALF)], src_t)
        pltpu.sync_copy(dst_hbm.at[pl.ds(off, _HALF)], dst_t)
        pltpu.async_copy(g_hbm.at[src_t.at[0]], r0, sem0)

        def pair(i, carry):
            pltpu.async_copy(g_hbm.at[src_t.at[2 * i + 1]], r1, sem1)
            pltpu.make_async_copy(g_hbm.at[src_t.at[0]], r0, sem0).wait()
            pltpu.sync_copy(r0, acc_sh.at[dst_t.at[2 * i]], add=True)

            @pl.when(i < _HALF // 2 - 1)
            def _():
                pltpu.async_copy(g_hbm.at[src_t.at[2 * i + 2]], r0, sem0)

            pltpu.make_async_copy(g_hbm.at[src_t.at[0]], r1, sem1).wait()
            pltpu.sync_copy(r1, acc_sh.at[dst_t.at[2 * i + 1]], add=True)
            return carry

        lax.fori_loop(0, _HALF // 2, pair, 0)

    plsc.subcore_barrier()
    pltpu.sync_copy(acc_sh.at[pl.ds(base, RPT)],
                    out_hbm.at[cid, pl.ds(base, RPT)])


@functools.cache
def _scat_call():
    return pl.kernel(
        _scat_body,
        out_type=jax.ShapeDtypeStruct((2, NP, 128), jnp.float32),
        mesh=_mesh(),
        scratch_types=[
            pltpu.VMEM_SHARED((NP, 128), jnp.float32),
            pltpu.VMEM((_HALF, 128), jnp.int32),
            pltpu.VMEM((_HALF, 128), jnp.int32),
            pltpu.VMEM((128, 128), jnp.float32),
            pltpu.VMEM((128, 128), jnp.float32),
            pltpu.SemaphoreType.DMA,
            pltpu.SemaphoreType.DMA,
        ],
    )


_RB = 640
_GRID = NP // _RB


def _t0_body(x_ref, w_ref, deg_ref, o_ref):
    dinv = lax.rsqrt(deg_ref[...] + 1.0)
    z = jnp.dot(x_ref[...], w_ref[...], preferred_element_type=jnp.float32)
    g = z * dinv
    o_ref[0] = g[:, :128]
    o_ref[1] = g[:, 128:]


def _t0(x, w, deg):
    return pl.pallas_call(
        _t0_body,
        grid=(_GRID,),
        in_specs=[
            pl.BlockSpec((_RB, D), lambda i: (i, 0)),
            pl.BlockSpec((D, D), lambda i: (0, 0)),
            pl.BlockSpec((_RB, 1), lambda i: (i, 0)),
        ],
        out_specs=pl.BlockSpec((2, _RB, 128), lambda i: (0, i, 0)),
        out_shape=jax.ShapeDtypeStruct((2, NP, 128), jnp.float32),
    )(x, w, deg)


def _t12_body(s_ref, g_ref, deg_ref, w_ref, b_ref, o_ref):
    dinv = lax.rsqrt(deg_ref[...] + 1.0)
    m = jnp.concatenate([s_ref[0] + g_ref[0], s_ref[1] + g_ref[1]], axis=1)
    h = jnp.maximum(m * dinv + b_ref[...], 0.0)
    z = jnp.dot(h, w_ref[...], preferred_element_type=jnp.float32)
    gn = z * dinv
    o_ref[0] = gn[:, :128]
    o_ref[1] = gn[:, 128:]


def _t12(s, g, deg, w, b):
    return pl.pallas_call(
        _t12_body,
        grid=(_GRID,),
        in_specs=[
            pl.BlockSpec((2, _RB, 128), lambda i: (0, i, 0)),
            pl.BlockSpec((2, _RB, 128), lambda i: (0, i, 0)),
            pl.BlockSpec((_RB, 1), lambda i: (i, 0)),
            pl.BlockSpec((D, D), lambda i: (0, 0)),
            pl.BlockSpec((1, D), lambda i: (0, 0)),
        ],
        out_specs=pl.BlockSpec((2, _RB, 128), lambda i: (0, i, 0)),
        out_shape=jax.ShapeDtypeStruct((2, NP, 128), jnp.float32),
    )(s, g, deg, w, b)


def _relu_feats(s_ref, g_ref, deg_ref, b_ref):
    dinv = lax.rsqrt(deg_ref[...] + 1.0)
    m = jnp.concatenate([s_ref[0] + g_ref[0], s_ref[1] + g_ref[1]], axis=1)
    return jnp.maximum(m * dinv + b_ref[...], 0.0)


def _t3_body(sp_ref, gp_ref, degp_ref, bp_ref, xbp_ref,
             sd_ref, gd_ref, degd_ref, bd_ref, xbd_ref,
             w0_ref, b0_ref, w1_ref, b1_ref, o_ref, macc):
    i = pl.program_id(0)

    @pl.when(i == 0)
    def _():
        macc[...] = jnp.full((B, 2 * D), -jnp.inf, jnp.float32)

    h3p = _relu_feats(sp_ref, gp_ref, degp_ref, bp_ref)
    h3d = _relu_feats(sd_ref, gd_ref, degd_ref, bd_ref)
    row = lax.broadcasted_iota(jnp.int32, (_RB, 1), 0) + i * _RB
    valid = row < N
    h3p = jnp.where(valid, h3p, -jnp.inf)
    h3d = jnp.where(valid, h3d, -jnp.inf)
    bidp = xbp_ref[...]
    bidd = xbd_ref[...]
    for b in range(B):
        pm = jnp.max(jnp.where(bidp == b, h3p, -jnp.inf), axis=0)
        dm = jnp.max(jnp.where(bidd == b, h3d, -jnp.inf), axis=0)
        macc[b, 0:D] = jnp.maximum(macc[b, 0:D], pm)
        macc[b, D:2 * D] = jnp.maximum(macc[b, D:2 * D], dm)

    @pl.when(i == _GRID - 1)
    def _():
        xx = macc[...]
        y = jnp.dot(xx, w0_ref[...], preferred_element_type=jnp.float32)
        y = y + b0_ref[...]
        o = jnp.dot(y, w1_ref[...], preferred_element_type=jnp.float32)
        o_ref[...] = o + b1_ref[...]


def _t3(sp, gp, degp, bp2, xbp, sd, gd, degd, bd2, xbd, lW0, lb0, lW1, lb1):
    blk = lambda i: (0, i, 0)
    row = lambda i: (i, 0)
    cst = lambda i: (0, 0)
    return pl.pallas_call(
        _t3_body,
        grid=(_GRID,),
        in_specs=[
            pl.BlockSpec((2, _RB, 128), blk),
            pl.BlockSpec((2, _RB, 128), blk),
            pl.BlockSpec((_RB, 1), row),
            pl.BlockSpec((1, D), cst),
            pl.BlockSpec((_RB, 1), row),
            pl.BlockSpec((2, _RB, 128), blk),
            pl.BlockSpec((2, _RB, 128), blk),
            pl.BlockSpec((_RB, 1), row),
            pl.BlockSpec((1, D), cst),
            pl.BlockSpec((_RB, 1), row),
            pl.BlockSpec((2 * D, 2 * D), cst),
            pl.BlockSpec((1, 2 * D), cst),
            pl.BlockSpec((2 * D, 1), cst),
            pl.BlockSpec((1, 1), cst),
        ],
        out_specs=pl.BlockSpec((B, 1), cst),
        out_shape=jax.ShapeDtypeStruct((B, 1), jnp.float32),
        scratch_shapes=[pltpu.VMEM((B, 2 * D), jnp.float32)],
    )(sp, gp, degp, bp2, xbp, sd, gd, degd, bd2, xbd, lW0, lb0, lW1, lb1)


def _prep_edges(edge_index):
    ei = edge_index.astype(jnp.int32)
    src = jnp.concatenate([ei[0], jnp.zeros((EP - E,), jnp.int32)])
    dst = jnp.concatenate([ei[1], jnp.full((EP - E,), N, jnp.int32)])
    src2 = jnp.stack([src, src + NP]).reshape(2, EROWS, 128)
    dst2 = dst.reshape(EROWS, 128)
    return src2, dst2


def kernel(x_p, x_d, edge_attr_p, edge_attr_d, x_p_batch, x_d_batch,
           edge_index_p, edge_index_d, Wp, bp, Wd, bd, lW0, lb0, lW1, lb1):
    src2p, dst2p = _prep_edges(edge_index_p)
    src2d, dst2d = _prep_edges(edge_index_d)
    z128 = jnp.zeros((RPT, 128), jnp.float32)
    ones128 = jnp.ones((128, 128), jnp.float32)

    degs = _deg_call()(dst2p, dst2d, z128, ones128)
    degp = degs[0, :N, 0:1]
    degd = degs[1, :N, 0:1]

    def branch(x, W, b, deg, src2, dst2):
        g = _t0(x, W[0], deg)
        s = _scat_call()(g.reshape(2 * NP, 128), src2, dst2, z128)
        g = _t12(s, g, deg, W[1], b[0].reshape(1, D))
        s = _scat_call()(g.reshape(2 * NP, 128), src2, dst2, z128)
        g = _t12(s, g, deg, W[2], b[1].reshape(1, D))
        s = _scat_call()(g.reshape(2 * NP, 128), src2, dst2, z128)
        return s, g

    sp, gp = branch(x_p, Wp, bp, degp, src2p, dst2p)
    sd, gd = branch(x_d, Wd, bd, degd, src2d, dst2d)

    return _t3(sp, gp, degp, bp[2].reshape(1, D),
               x_p_batch.astype(jnp.int32).reshape(N, 1),
               sd, gd, degd, bd[2].reshape(1, D),
               x_d_batch.astype(jnp.int32).reshape(N, 1),
               lW0, lb0.reshape(1, 2 * D), lW1, lb1.reshape(1, 1))
